# baseline (speedup 1.0000x reference)
import functools

import jax
import jax.numpy as jnp
from jax import lax
from jax.experimental import pallas as pl
from jax.experimental.pallas import tpu as pltpu
from jax.experimental.pallas import tpu_sc as plsc

N = 10000
D = 128
E = 320000

NC = 2
NS = 16
DH = D // NC
CHUNK = 128
NCHUNK = 160
E_PAD = NS * NCHUNK * CHUNK
N_PAD = 10112
SLICE = N_PAD // NS
ZROWS = 128
_PIECES = [(0, 128), (128, 128), (256, 128), (384, 128), (512, 120)]


def _sc_agg_body(x_hbm, pk_hbm, p_hbm, c_hbm,
                 pk_v, src_q, dst_q, rows0, rows1, rows2, rows3,
                 ones_v, zbuf, cbuf, acc, cacc, sg, *, with_counts):
    c_idx = lax.axis_index("core")
    s_idx = lax.axis_index("subcore")
    row0 = s_idx * SLICE

    z16 = jnp.zeros((16,), jnp.float32)
    o16 = jnp.ones((16,), jnp.float32)

    @pl.loop(0, ZROWS)
    def _(i):
        for j in range(DH // 16):
            zbuf[i, pl.ds(j * 16, 16)] = z16

    if with_counts:
        @pl.loop(0, SLICE, step=16)
        def _(i):
            cbuf[pl.ds(i, 16)] = z16

        for j in range(CHUNK // 16):
            ones_v[pl.ds(j * 16, 16)] = o16

    for off, ln in _PIECES:
        pltpu.sync_copy(zbuf.at[pl.ds(0, ln)], acc.at[pl.ds(row0 + off, ln)])
    if with_counts:
        pltpu.sync_copy(cbuf, cacc.at[pl.ds(row0, SLICE)])

    pltpu.sync_copy(pk_hbm.at[s_idx], pk_v)

    plsc.subcore_barrier()

    xc = x_hbm.at[c_idx]

    bufs = [(0, rows0), (1, rows1), (2, rows2), (3, rows3)]

    def issue(j, b):
        r, rows = b
        for k in range(CHUNK // 16):
            e = pk_v[j, pl.ds(k * 16, 16)]
            src_q[r, pl.ds(k * 16, 16)] = e & 0x3FFF
            dst_q[r, pl.ds(k * 16, 16)] = lax.shift_right_logical(e, 14)
        pltpu.async_copy(xc.at[src_q.at[r]], rows, sg)

    def scat(b):
        r, rows = b
        pltpu.make_async_copy(xc.at[src_q.at[r]], rows, sg).wait()
        pltpu.sync_copy(rows, acc.at[dst_q.at[r]], add=True)
        if with_counts:
            pltpu.sync_copy(ones_v, cacc.at[dst_q.at[r]], add=True)

    @pl.loop(0, NCHUNK + 4, step=4)
    def _(j):
        for r in range(4):
            c = j + r

            @pl.when(c < NCHUNK)
            def _():
                issue(c, bufs[r])

            @pl.when(jnp.logical_and(c >= 2, c < NCHUNK + 2))
            def _():
                scat(bufs[(r + 2) % 4])

    plsc.subcore_barrier()

    for off, ln in _PIECES:
        pltpu.sync_copy(acc.at[pl.ds(row0 + off, ln)], zbuf.at[pl.ds(0, ln)])
        pltpu.sync_copy(zbuf.at[pl.ds(0, ln)],
                        p_hbm.at[c_idx].at[pl.ds(row0 + off, ln)])
    if with_counts:
        pltpu.sync_copy(cacc.at[pl.ds(row0, SLICE)], cbuf)
        pltpu.sync_copy(cbuf, c_hbm.at[c_idx].at[pl.ds(row0, SLICE)])


def _sc_aggregate(xsplit, packed, with_counts):
    mesh = plsc.VectorSubcoreMesh(core_axis_name="core", subcore_axis_name="subcore")
    k = functools.partial(
        pl.kernel,
        out_type=(jax.ShapeDtypeStruct((NC, N_PAD, DH), jnp.float32),
                  jax.ShapeDtypeStruct((NC, N_PAD), jnp.float32)),
        mesh=mesh,
        scratch_types=(
            [pltpu.VMEM((NCHUNK, CHUNK), jnp.int32)]
            + [pltpu.VMEM((4, CHUNK), jnp.int32)] * 2
            + [pltpu.VMEM((CHUNK, DH), jnp.float32)] * 4
            + [
                pltpu.VMEM((CHUNK,), jnp.float32),
                pltpu.VMEM((ZROWS, DH), jnp.float32),
                pltpu.VMEM((SLICE,), jnp.float32),
                pltpu.VMEM_SHARED((N_PAD, DH), jnp.float32),
                pltpu.VMEM_SHARED((N_PAD,), jnp.float32),
            ]
            + [pltpu.SemaphoreType.DMA]
        ),
        compiler_params=pltpu.CompilerParams(use_tc_tiling_on_sc=False),
    )(functools.partial(_sc_agg_body, with_counts=with_counts))
    return k(xsplit, packed)


def _tc_body(p_ref, c_ref, x_ref, wl_ref, bl_ref, wr_ref, out_ref, *, sigmoid):
    cnt = jnp.clip(c_ref[0], 1.0, None)
    agg = jnp.concatenate([p_ref[0], p_ref[1]], axis=1)
    xin = jnp.concatenate([x_ref[0], x_ref[1]], axis=1)
    mean = agg / cnt
    r = (jnp.dot(mean, wl_ref[...], preferred_element_type=jnp.float32)
         + bl_ref[...]
         + jnp.dot(xin, wr_ref[...], preferred_element_type=jnp.float32))
    if sigmoid:
        out_ref[...] = jax.nn.sigmoid(r)
    else:
        out_ref[0] = r[:, :DH]
        out_ref[1] = r[:, DH:]


_TC_B = 2000


def _tc_layer(p, cnts, xin, wlT, bl2d, wrT, sigmoid):
    B = _TC_B
    out_shape = (jax.ShapeDtypeStruct((N, D), jnp.float32) if sigmoid else
                 jax.ShapeDtypeStruct((NC, N, DH), jnp.float32))
    out_spec = (pl.BlockSpec((B, D), lambda i: (i, 0)) if sigmoid else
                pl.BlockSpec((NC, B, DH), lambda i: (0, i, 0)))
    return pl.pallas_call(
        functools.partial(_tc_body, sigmoid=sigmoid),
        grid=(N // B,),
        in_specs=[
            pl.BlockSpec((NC, B, DH), lambda i: (0, i, 0)),
            pl.BlockSpec((NC, B, 1), lambda i: (0, i, 0)),
            pl.BlockSpec((NC, B, DH), lambda i: (0, i, 0)),
            pl.BlockSpec((D, D), lambda i: (0, 0)),
            pl.BlockSpec((1, D), lambda i: (0, 0)),
            pl.BlockSpec((D, D), lambda i: (0, 0)),
        ],
        out_specs=out_spec,
        out_shape=out_shape,
    )(p, cnts, xin, wlT, bl2d, wrT)


def kernel(x, edge_index, h, Wl1, bl1, Wr1, Wl2, bl2, Wr2):
    del h
    src = edge_index[0].astype(jnp.int32)
    dst = edge_index[1].astype(jnp.int32)
    pad = E_PAD - E
    packed = (dst << 14) | src
    packed = jnp.concatenate([packed, jnp.full((pad,), N << 14, jnp.int32)])
    packed = packed.reshape(NS, NCHUNK, CHUNK)
    packed = jnp.concatenate([packed, jnp.zeros((3 * NS, NCHUNK, CHUNK), jnp.int32)])

    xsplit = x.reshape(N, NC, DH).transpose(1, 0, 2)

    packed, xsplit = lax.optimization_barrier((packed, xsplit))

    p1, c1 = _sc_aggregate(xsplit, packed, with_counts=True)
    cnts = c1[:, :, None]
    out1 = _tc_layer(p1, cnts, xsplit, Wl1.T, bl1[None, :], Wr1.T, sigmoid=False)
    p2, _ = _sc_aggregate(out1, packed, with_counts=True)
    out2 = _tc_layer(p2, cnts, out1, Wl2.T, bl2[None, :], Wr2.T, sigmoid=True)
    return out2


# speedup vs baseline: 1.0465x; 1.0465x over previous
import functools

import jax
import jax.numpy as jnp
from jax import lax
from jax.experimental import pallas as pl
from jax.experimental.pallas import tpu as pltpu
from jax.experimental.pallas import tpu_sc as plsc

N = 10000
D = 128
E = 320000

NC = 2
NS = 16
DH = D // NC
CHUNK = 128
NCHUNK = 160
E_PAD = NS * NCHUNK * CHUNK
N_PAD = 10112
SLICE = N_PAD // NS
ZROWS = 128
_PIECES = [(0, 128), (128, 128), (256, 128), (384, 128), (512, 120)]


def _sc_agg_body(x_hbm, pk_hbm, p_hbm, c_hbm,
                 pk_v, src_q, dst_q, rows0, rows1, rows2, rows3,
                 ones_v, zbuf, cbuf, acc, cacc, sg, ss, *, with_counts):
    c_idx = lax.axis_index("core")
    s_idx = lax.axis_index("subcore")
    row0 = s_idx * SLICE

    z16 = jnp.zeros((16,), jnp.float32)
    o16 = jnp.ones((16,), jnp.float32)

    @pl.loop(0, ZROWS)
    def _(i):
        for j in range(DH // 16):
            zbuf[i, pl.ds(j * 16, 16)] = z16

    if with_counts:
        @pl.loop(0, SLICE, step=16)
        def _(i):
            cbuf[pl.ds(i, 16)] = z16

        for j in range(CHUNK // 16):
            ones_v[pl.ds(j * 16, 16)] = o16

    for off, ln in _PIECES:
        pltpu.sync_copy(zbuf.at[pl.ds(0, ln)], acc.at[pl.ds(row0 + off, ln)])
    if with_counts:
        pltpu.sync_copy(cbuf, cacc.at[pl.ds(row0, SLICE)])

    pltpu.sync_copy(pk_hbm.at[s_idx], pk_v)

    plsc.subcore_barrier()

    xc = x_hbm.at[c_idx]

    bufs = [(0, rows0), (1, rows1), (2, rows2), (3, rows3)]

    def issue(j, b):
        r, rows = b
        for k in range(CHUNK // 16):
            e = pk_v[j, pl.ds(k * 16, 16)]
            src_q[r, pl.ds(k * 16, 16)] = e & 0x3FFF
            dst_q[r, pl.ds(k * 16, 16)] = lax.shift_right_logical(e, 14)
        pltpu.async_copy(xc.at[src_q.at[r]], rows, sg)

    def scat(b):
        r, rows = b
        pltpu.make_async_copy(xc.at[src_q.at[r]], rows, sg).wait()
        pltpu.async_copy(rows, acc.at[dst_q.at[r]], ss, add=True)
        if with_counts:
            pltpu.sync_copy(ones_v, cacc.at[dst_q.at[r]], add=True)

    def wait_scat(b):
        r, rows = b
        pltpu.make_async_copy(rows, acc.at[dst_q.at[r]], ss).wait()

    issue(0, bufs[0])
    issue(1, bufs[1])
    scat(bufs[0])
    issue(2, bufs[2])
    scat(bufs[1])
    issue(3, bufs[3])

    @pl.loop(4, NCHUNK, step=4)
    def _(j):
        for r in range(4):
            wait_scat(bufs[r])
            issue(j + r, bufs[r])
            scat(bufs[(r + 2) % 4])

    scat(bufs[2])
    scat(bufs[3])
    for r in range(4):
        wait_scat(bufs[r])

    plsc.subcore_barrier()

    for off, ln in _PIECES:
        pltpu.sync_copy(acc.at[pl.ds(row0 + off, ln)], zbuf.at[pl.ds(0, ln)])
        pltpu.sync_copy(zbuf.at[pl.ds(0, ln)],
                        p_hbm.at[c_idx].at[pl.ds(row0 + off, ln)])
    if with_counts:
        pltpu.sync_copy(cacc.at[pl.ds(row0, SLICE)], cbuf)
        pltpu.sync_copy(cbuf, c_hbm.at[c_idx].at[pl.ds(row0, SLICE)])


def _sc_aggregate(xsplit, packed, with_counts):
    mesh = plsc.VectorSubcoreMesh(core_axis_name="core", subcore_axis_name="subcore")
    k = functools.partial(
        pl.kernel,
        out_type=(jax.ShapeDtypeStruct((NC, N_PAD, DH), jnp.float32),
                  jax.ShapeDtypeStruct((NC, N_PAD), jnp.float32)),
        mesh=mesh,
        scratch_types=(
            [pltpu.VMEM((NCHUNK, CHUNK), jnp.int32)]
            + [pltpu.VMEM((4, CHUNK), jnp.int32)] * 2
            + [pltpu.VMEM((CHUNK, DH), jnp.float32)] * 4
            + [
                pltpu.VMEM((CHUNK,), jnp.float32),
                pltpu.VMEM((ZROWS, DH), jnp.float32),
                pltpu.VMEM((SLICE,), jnp.float32),
                pltpu.VMEM_SHARED((N_PAD, DH), jnp.float32),
                pltpu.VMEM_SHARED((N_PAD,), jnp.float32),
            ]
            + [pltpu.SemaphoreType.DMA] * 2
        ),
        compiler_params=pltpu.CompilerParams(use_tc_tiling_on_sc=False),
    )(functools.partial(_sc_agg_body, with_counts=with_counts))
    return k(xsplit, packed)


def _tc_body(p_ref, c_ref, x_ref, wl_ref, bl_ref, wr_ref, out_ref, *, sigmoid):
    cnt = jnp.clip(c_ref[0], 1.0, None)
    agg = jnp.concatenate([p_ref[0], p_ref[1]], axis=1)
    xin = jnp.concatenate([x_ref[0], x_ref[1]], axis=1)
    mean = agg / cnt
    r = (jnp.dot(mean, wl_ref[...], preferred_element_type=jnp.float32)
         + bl_ref[...]
         + jnp.dot(xin, wr_ref[...], preferred_element_type=jnp.float32))
    if sigmoid:
        out_ref[...] = jax.nn.sigmoid(r)
    else:
        out_ref[0] = r[:, :DH]
        out_ref[1] = r[:, DH:]


_TC_B = 2000


def _tc_layer(p, cnts, xin, wlT, bl2d, wrT, sigmoid):
    B = _TC_B
    out_shape = (jax.ShapeDtypeStruct((N, D), jnp.float32) if sigmoid else
                 jax.ShapeDtypeStruct((NC, N, DH), jnp.float32))
    out_spec = (pl.BlockSpec((B, D), lambda i: (i, 0)) if sigmoid else
                pl.BlockSpec((NC, B, DH), lambda i: (0, i, 0)))
    return pl.pallas_call(
        functools.partial(_tc_body, sigmoid=sigmoid),
        grid=(N // B,),
        in_specs=[
            pl.BlockSpec((NC, B, DH), lambda i: (0, i, 0)),
            pl.BlockSpec((NC, B, 1), lambda i: (0, i, 0)),
            pl.BlockSpec((NC, B, DH), lambda i: (0, i, 0)),
            pl.BlockSpec((D, D), lambda i: (0, 0)),
            pl.BlockSpec((1, D), lambda i: (0, 0)),
            pl.BlockSpec((D, D), lambda i: (0, 0)),
        ],
        out_specs=out_spec,
        out_shape=out_shape,
    )(p, cnts, xin, wlT, bl2d, wrT)


def kernel(x, edge_index, h, Wl1, bl1, Wr1, Wl2, bl2, Wr2):
    del h
    src = edge_index[0].astype(jnp.int32)
    dst = edge_index[1].astype(jnp.int32)
    pad = E_PAD - E
    packed = (dst << 14) | src
    packed = jnp.concatenate([packed, jnp.full((pad,), N << 14, jnp.int32)])
    packed = packed.reshape(NS, NCHUNK, CHUNK)
    packed = jnp.concatenate([packed, jnp.zeros((3 * NS, NCHUNK, CHUNK), jnp.int32)])

    xsplit = x.reshape(N, NC, DH).transpose(1, 0, 2)

    packed, xsplit = lax.optimization_barrier((packed, xsplit))

    p1, c1 = _sc_aggregate(xsplit, packed, with_counts=True)
    cnts = c1[:, :, None]
    out1 = _tc_layer(p1, cnts, xsplit, Wl1.T, bl1[None, :], Wr1.T, sigmoid=False)
    p2, _ = _sc_aggregate(out1, packed, with_counts=False)
    out2 = _tc_layer(p2, cnts, out1, Wl2.T, bl2[None, :], Wr2.T, sigmoid=True)
    return out2


# speedup vs baseline: 1.3651x; 1.3044x over previous
import functools

import jax
import jax.numpy as jnp
from jax import lax
from jax.experimental import pallas as pl
from jax.experimental.pallas import tpu as pltpu
from jax.experimental.pallas import tpu_sc as plsc

N = 10000
D = 128
E = 320000

NC = 2
NS = 16
DH = D // NC
CHUNK = 128
NCHUNK = 158
E_PAD = NS * NCHUNK * CHUNK
N_PAD = 10112
SLICE = N_PAD // NS


def _sc_agg_body(x_hbm, pk_hbm, p_hbm, c_hbm,
                 pk_v, src_q, dst_q, rows0, rows1,
                 ones_v, zbuf, cbuf, acc, cacc, sg, ss, *, with_counts):
    c_idx = lax.axis_index("core")
    s_idx = lax.axis_index("subcore")
    row0 = s_idx * SLICE

    z16 = jnp.zeros((16,), jnp.float32)
    o16 = jnp.ones((16,), jnp.float32)

    @pl.loop(0, SLICE)
    def _(i):
        for j in range(DH // 16):
            zbuf[i, pl.ds(j * 16, 16)] = z16

    if with_counts:
        @pl.loop(0, SLICE, step=16)
        def _(i):
            cbuf[pl.ds(i, 16)] = z16

        for j in range(CHUNK // 16):
            ones_v[pl.ds(j * 16, 16)] = o16

    pltpu.sync_copy(zbuf, acc.at[pl.ds(row0, SLICE)])
    if with_counts:
        pltpu.sync_copy(cbuf, cacc.at[pl.ds(row0, SLICE)])

    pltpu.sync_copy(pk_hbm.at[s_idx], pk_v)

    plsc.subcore_barrier()

    xc = x_hbm.at[c_idx]

    def gather(j, r, rows, sem):
        for k in range(CHUNK // 16):
            e = pk_v[j, pl.ds(k * 16, 16)]
            src_q[r, pl.ds(k * 16, 16)] = e & 0x3FFF
            dst_q[r, pl.ds(k * 16, 16)] = lax.shift_right_logical(e, 14)
        pltpu.async_copy(xc.at[src_q.at[r]], rows, sem)

    def consume(r, rows, sem):
        pltpu.make_async_copy(xc.at[src_q.at[r]], rows, sem).wait()
        pltpu.sync_copy(rows, acc.at[dst_q.at[r]], add=True)
        if with_counts:
            pltpu.sync_copy(ones_v, cacc.at[dst_q.at[r]], add=True)

    gather(0, 0, rows0, sg)

    @pl.loop(0, NCHUNK - 2, step=2)
    def _(j):
        gather(j + 1, 1, rows1, ss)
        consume(0, rows0, sg)
        gather(j + 2, 0, rows0, sg)
        consume(1, rows1, ss)

    gather(NCHUNK - 1, 1, rows1, ss)
    consume(0, rows0, sg)
    consume(1, rows1, ss)

    plsc.subcore_barrier()

    pltpu.sync_copy(acc.at[pl.ds(row0, SLICE)], zbuf)
    pltpu.sync_copy(zbuf, p_hbm.at[c_idx].at[pl.ds(row0, SLICE)])
    if with_counts:
        pltpu.sync_copy(cacc.at[pl.ds(row0, SLICE)], cbuf)
        pltpu.sync_copy(cbuf, c_hbm.at[c_idx].at[pl.ds(row0, SLICE)])


def _sc_aggregate(xsplit, packed, with_counts):
    mesh = plsc.VectorSubcoreMesh(core_axis_name="core", subcore_axis_name="subcore")
    k = functools.partial(
        pl.kernel,
        out_type=(jax.ShapeDtypeStruct((NC, N_PAD, DH), jnp.float32),
                  jax.ShapeDtypeStruct((NC, N_PAD), jnp.float32)),
        mesh=mesh,
        scratch_types=(
            [pltpu.VMEM((NCHUNK, CHUNK), jnp.int32)]
            + [pltpu.VMEM((2, CHUNK), jnp.int32)] * 2
            + [pltpu.VMEM((CHUNK, DH), jnp.float32)] * 2
            + [
                pltpu.VMEM((CHUNK,), jnp.float32),
                pltpu.VMEM((SLICE, DH), jnp.float32),
                pltpu.VMEM((SLICE,), jnp.float32),
                pltpu.VMEM_SHARED((N_PAD, DH), jnp.float32),
                pltpu.VMEM_SHARED((N_PAD,), jnp.float32),
            ]
            + [pltpu.SemaphoreType.DMA] * 2
        ),
        compiler_params=pltpu.CompilerParams(use_tc_tiling_on_sc=False),
    )(functools.partial(_sc_agg_body, with_counts=with_counts))
    return k(xsplit, packed)


def _tc_body(p_ref, c_ref, x_ref, wl_ref, bl_ref, wr_ref, out_ref, *, sigmoid):
    cnt = jnp.clip(c_ref[0], 1.0, None)
    agg = jnp.concatenate([p_ref[0], p_ref[1]], axis=1)
    xin = jnp.concatenate([x_ref[0], x_ref[1]], axis=1)
    mean = agg / cnt
    r = (jnp.dot(mean, wl_ref[...], preferred_element_type=jnp.float32)
         + bl_ref[...]
         + jnp.dot(xin, wr_ref[...], preferred_element_type=jnp.float32))
    if sigmoid:
        out_ref[...] = jax.nn.sigmoid(r)
    else:
        out_ref[0] = r[:, :DH]
        out_ref[1] = r[:, DH:]


_TC_B = 2000


def _tc_layer(p, cnts, xin, wlT, bl2d, wrT, sigmoid):
    B = _TC_B
    out_shape = (jax.ShapeDtypeStruct((N, D), jnp.float32) if sigmoid else
                 jax.ShapeDtypeStruct((NC, N, DH), jnp.float32))
    out_spec = (pl.BlockSpec((B, D), lambda i: (i, 0)) if sigmoid else
                pl.BlockSpec((NC, B, DH), lambda i: (0, i, 0)))
    return pl.pallas_call(
        functools.partial(_tc_body, sigmoid=sigmoid),
        grid=(N // B,),
        in_specs=[
            pl.BlockSpec((NC, B, DH), lambda i: (0, i, 0)),
            pl.BlockSpec((NC, B, 1), lambda i: (0, i, 0)),
            pl.BlockSpec((NC, B, DH), lambda i: (0, i, 0)),
            pl.BlockSpec((D, D), lambda i: (0, 0)),
            pl.BlockSpec((1, D), lambda i: (0, 0)),
            pl.BlockSpec((D, D), lambda i: (0, 0)),
        ],
        out_specs=out_spec,
        out_shape=out_shape,
    )(p, cnts, xin, wlT, bl2d, wrT)


def kernel(x, edge_index, h, Wl1, bl1, Wr1, Wl2, bl2, Wr2):
    del h
    src = edge_index[0].astype(jnp.int32)
    dst = edge_index[1].astype(jnp.int32)
    pad = E_PAD - E
    packed = (dst << 14) | src
    packed = jnp.concatenate([packed, jnp.full((pad,), N << 14, jnp.int32)])
    packed = packed.reshape(NS, NCHUNK, CHUNK)

    xsplit = x.reshape(N, NC, DH).transpose(1, 0, 2)

    packed, xsplit = lax.optimization_barrier((packed, xsplit))

    p1, c1 = _sc_aggregate(xsplit, packed, with_counts=True)
    cnts = c1[:, :, None]
    out1 = _tc_layer(p1, cnts, xsplit, Wl1.T, bl1[None, :], Wr1.T, sigmoid=False)
    p2, _ = _sc_aggregate(out1, packed, with_counts=False)
    out2 = _tc_layer(p2, cnts, out1, Wl2.T, bl2[None, :], Wr2.T, sigmoid=True)
    return out2
